# baseline (speedup 1.0000x reference)
import functools

import jax
import jax.numpy as jnp
from jax import lax
from jax.experimental import pallas as pl
from jax.experimental.pallas import tpu as pltpu
from jax.experimental.pallas import tpu_sc as plsc

D_E = 128
NUM_WORKERS = 32


def _sc_gather(idx3, table, per_w, seq):
    n_batch = NUM_WORKERS * per_w
    mesh = plsc.VectorSubcoreMesh(core_axis_name="c", subcore_axis_name="s")

    @functools.partial(
        pl.kernel,
        out_type=jax.ShapeDtypeStruct((n_batch, seq, D_E), jnp.float32),
        mesh=mesh,
        compiler_params=pltpu.CompilerParams(use_tc_tiling_on_sc=True),
        scratch_types=[
            pltpu.VMEM((per_w, seq), jnp.int32),
            pltpu.VMEM((2, seq, D_E), jnp.float32),
            pltpu.SemaphoreType.DMA,
            pltpu.SemaphoreType.DMA,
        ],
    )
    def k(idx_hbm, table_hbm, out_hbm, idx_v, rows_v, g0, g1):
        assert per_w % 2 == 0
        wid = lax.axis_index("s") * 2 + lax.axis_index("c")
        base = wid * per_w
        pltpu.sync_copy(idx_hbm.at[wid], idx_v)

        pltpu.async_copy(table_hbm.at[idx_v.at[0]], rows_v.at[0], g0)

        def body(i, _):
            b = i * 2
            pltpu.async_copy(table_hbm.at[idx_v.at[b + 1]], rows_v.at[1], g1)
            pltpu.make_async_copy(
                table_hbm.at[idx_v.at[b]], rows_v.at[0], g0
            ).wait()
            pltpu.sync_copy(rows_v.at[0], out_hbm.at[base + b])

            @pl.when(b + 2 < per_w)
            def _():
                pltpu.async_copy(
                    table_hbm.at[idx_v.at[b + 2]], rows_v.at[0], g0
                )

            pltpu.make_async_copy(
                table_hbm.at[idx_v.at[b + 1]], rows_v.at[1], g1
            ).wait()
            pltpu.sync_copy(rows_v.at[1], out_hbm.at[base + b + 1])
            return 0

        lax.fori_loop(0, per_w // 2, body, 0, unroll=False)

    return k(idx3, table)


def kernel(input_ids, table):
    b, s = input_ids.shape
    per_w = b // NUM_WORKERS
    assert per_w * NUM_WORKERS == b
    idx3 = input_ids.reshape(NUM_WORKERS, per_w, s).astype(jnp.int32)
    return _sc_gather(idx3, table, per_w, s)
